# initial kernel scaffold (unmeasured)
import jax
import jax.numpy as jnp
from jax import lax
from jax.experimental import pallas as pl
from jax.experimental.pallas import tpu as pltpu


def kernel(
    x,
):
    def body(*refs):
        pass

    out_shape = jax.ShapeDtypeStruct(..., jnp.float32)
    return pl.pallas_call(body, out_shape=out_shape)(...)


# baseline (device time: 14400 ns/iter reference)
import jax
import jax.numpy as jnp
from jax import lax
from jax.experimental import pallas as pl
from jax.experimental.pallas import tpu as pltpu

N_DEV = 4


def kernel(x):
    m_per, n = x.shape

    def body(x_ref, out_ref, comm_ref, send_sems, recv_sems):
        my_pos = lax.axis_index("i")

        xv = x_ref[:, :].astype(jnp.float32)
        mval = jnp.max(xv, axis=0, keepdims=True)
        rows = lax.broadcasted_iota(jnp.int32, (m_per, n), 0)
        lidx = jnp.min(
            jnp.where(xv == mval, rows, m_per), axis=0, keepdims=True
        )
        gidx = (lidx + my_pos * m_per).astype(jnp.float32)
        comm_ref[0, :, :] = jnp.concatenate([mval, gidx], axis=0)

        rdmas = []
        for d in range(1, N_DEV):
            peer = lax.rem(my_pos + d, N_DEV)
            rdma = pltpu.make_async_remote_copy(
                src_ref=comm_ref.at[0],
                dst_ref=comm_ref.at[d],
                send_sem=send_sems.at[d],
                recv_sem=recv_sems.at[d],
                device_id=(peer,),
                device_id_type=pl.DeviceIdType.MESH,
            )
            rdma.start()
            rdmas.append(rdma)
        for rdma in rdmas:
            rdma.wait()

        bv = comm_ref[0, 0:1, :]
        bi = comm_ref[0, 1:2, :]
        for d in range(1, N_DEV):
            v = comm_ref[d, 0:1, :]
            i = comm_ref[d, 1:2, :]
            take = (v > bv) | ((v == bv) & (i < bi))
            bv = jnp.where(take, v, bv)
            bi = jnp.where(take, i, bi)
        out_ref[0:1, :] = bv
        out_ref[1:2, :] = bi

    return pl.pallas_call(
        body,
        out_shape=jax.ShapeDtypeStruct((2, n), jnp.float32),
        in_specs=[pl.BlockSpec(memory_space=pltpu.VMEM)],
        out_specs=pl.BlockSpec(memory_space=pltpu.VMEM),
        scratch_shapes=[
            pltpu.VMEM((N_DEV, 2, n), jnp.float32),
            pltpu.SemaphoreType.DMA((N_DEV,)),
            pltpu.SemaphoreType.DMA((N_DEV,)),
        ],
    )(x)


# device time: 14145 ns/iter; 1.0180x vs baseline; 1.0180x over previous
import jax
import jax.numpy as jnp
from jax import lax
from jax.experimental import pallas as pl
from jax.experimental.pallas import tpu as pltpu

N_DEV = 4
CHUNK = 256


def kernel(x):
    m_per, n = x.shape
    n_chunks = m_per // CHUNK

    def body(x_ref, out_ref, acc_ref, comm_ref, send_sems, recv_sems):
        g = pl.program_id(0)
        my_pos = lax.axis_index("i")

        xv = x_ref[:, :].astype(jnp.float32)
        mval = jnp.max(xv, axis=0, keepdims=True)
        rows = lax.broadcasted_iota(jnp.int32, (CHUNK, n), 0)
        lidx = jnp.min(
            jnp.where(xv == mval, rows, CHUNK), axis=0, keepdims=True
        )
        gidx = (lidx + g * CHUNK + my_pos * m_per).astype(jnp.float32)

        @pl.when(g == 0)
        def _():
            acc_ref[0:1, :] = mval
            acc_ref[1:2, :] = gidx

        @pl.when(g > 0)
        def _():
            bv = acc_ref[0:1, :]
            take = mval > bv
            acc_ref[0:1, :] = jnp.where(take, mval, bv)
            acc_ref[1:2, :] = jnp.where(take, gidx, acc_ref[1:2, :])

        @pl.when(g == n_chunks - 1)
        def _():
            comm_ref[0, :, :] = acc_ref[:, :]
            rdmas = []
            for d in range(1, N_DEV):
                peer = lax.rem(my_pos + d, N_DEV)
                rdma = pltpu.make_async_remote_copy(
                    src_ref=comm_ref.at[0],
                    dst_ref=comm_ref.at[d],
                    send_sem=send_sems.at[d],
                    recv_sem=recv_sems.at[d],
                    device_id=(peer,),
                    device_id_type=pl.DeviceIdType.MESH,
                )
                rdma.start()
                rdmas.append(rdma)
            for rdma in rdmas:
                rdma.wait()

            bv = comm_ref[0, 0:1, :]
            bi = comm_ref[0, 1:2, :]
            for d in range(1, N_DEV):
                v = comm_ref[d, 0:1, :]
                i = comm_ref[d, 1:2, :]
                take = (v > bv) | ((v == bv) & (i < bi))
                bv = jnp.where(take, v, bv)
                bi = jnp.where(take, i, bi)
            out_ref[0:1, :] = bv
            out_ref[1:2, :] = bi

    return pl.pallas_call(
        body,
        grid=(n_chunks,),
        out_shape=jax.ShapeDtypeStruct((2, n), jnp.float32),
        in_specs=[
            pl.BlockSpec((CHUNK, n), lambda g: (g, 0), memory_space=pltpu.VMEM)
        ],
        out_specs=pl.BlockSpec((2, n), lambda g: (0, 0), memory_space=pltpu.VMEM),
        scratch_shapes=[
            pltpu.VMEM((2, n), jnp.float32),
            pltpu.VMEM((N_DEV, 2, n), jnp.float32),
            pltpu.SemaphoreType.DMA((N_DEV,)),
            pltpu.SemaphoreType.DMA((N_DEV,)),
        ],
        compiler_params=pltpu.CompilerParams(
            dimension_semantics=("arbitrary",),
        ),
    )(x)


# device time: 10083 ns/iter; 1.4281x vs baseline; 1.4029x over previous
import jax
import jax.numpy as jnp
from jax import lax
from jax.experimental import pallas as pl
from jax.experimental.pallas import tpu as pltpu

N_DEV = 4
CHUNK = 1024


def kernel(x):
    m_per, n = x.shape
    n_chunks = m_per // CHUNK

    def body(x_ref, out_ref, comm_ref, send_sems, recv_sems):
        g = pl.program_id(0)
        my_pos = lax.axis_index("i")
        barrier_sem = pltpu.get_barrier_semaphore()

        @pl.when(g == 0)
        def _():
            for d in range(1, N_DEV):
                pl.semaphore_signal(
                    barrier_sem,
                    inc=1,
                    device_id=(lax.rem(my_pos + d, N_DEV),),
                    device_id_type=pl.DeviceIdType.MESH,
                )

        xv = x_ref[:, :].astype(jnp.float32)
        mval = jnp.max(xv, axis=0, keepdims=True)
        lidx = jnp.argmax(xv, axis=0).reshape(1, n).astype(jnp.int32)
        gidx = (lidx + g * CHUNK + my_pos * m_per).astype(jnp.float32)

        @pl.when(g == 0)
        def _():
            comm_ref[0, 0:1, :] = mval
            comm_ref[0, 1:2, :] = gidx

        @pl.when(g > 0)
        def _():
            bv = comm_ref[0, 0:1, :]
            take = mval > bv
            comm_ref[0, 0:1, :] = jnp.where(take, mval, bv)
            comm_ref[0, 1:2, :] = jnp.where(take, gidx, comm_ref[0, 1:2, :])

        @pl.when(g == n_chunks - 1)
        def _():
            pl.semaphore_wait(barrier_sem, N_DEV - 1)
            rdmas = []
            for d in range(1, N_DEV):
                peer = lax.rem(my_pos + d, N_DEV)
                rdma = pltpu.make_async_remote_copy(
                    src_ref=comm_ref.at[0],
                    dst_ref=comm_ref.at[d],
                    send_sem=send_sems.at[d],
                    recv_sem=recv_sems.at[d],
                    device_id=(peer,),
                    device_id_type=pl.DeviceIdType.MESH,
                )
                rdma.start()
                rdmas.append(rdma)
            for rdma in rdmas:
                rdma.wait_recv()

            bv = comm_ref[0, 0:1, :]
            bi = comm_ref[0, 1:2, :]
            for d in range(1, N_DEV):
                v = comm_ref[d, 0:1, :]
                i = comm_ref[d, 1:2, :]
                take = (v > bv) | ((v == bv) & (i < bi))
                bv = jnp.where(take, v, bv)
                bi = jnp.where(take, i, bi)
            out_ref[0:1, :] = bv
            out_ref[1:2, :] = bi

            for rdma in rdmas:
                rdma.wait_send()

    return pl.pallas_call(
        body,
        grid=(n_chunks,),
        out_shape=jax.ShapeDtypeStruct((2, n), jnp.float32),
        in_specs=[
            pl.BlockSpec((CHUNK, n), lambda g: (g, 0), memory_space=pltpu.VMEM)
        ],
        out_specs=pl.BlockSpec((2, n), lambda g: (0, 0), memory_space=pltpu.VMEM),
        scratch_shapes=[
            pltpu.VMEM((N_DEV, 2, n), jnp.float32),
            pltpu.SemaphoreType.DMA((N_DEV,)),
            pltpu.SemaphoreType.DMA((N_DEV,)),
        ],
        compiler_params=pltpu.CompilerParams(
            dimension_semantics=("arbitrary",),
            collective_id=0,
        ),
    )(x)
